# baseline (speedup 1.0000x reference)
import functools

import jax
import jax.numpy as jnp
from jax import lax
from jax.experimental import pallas as pl
from jax.experimental.pallas import tpu as pltpu
from jax.experimental.pallas import tpu_sc as plsc

_L = 16
_C = 128
_NW = 32
_R = 32
_CHUNK = _R * _C
_NB = 8


def _sc_permute(xf, indexes, n_rows):
    rows_per_w = n_rows // _NW
    n_chunks = rows_per_w // _R
    n_groups = n_chunks // _NB
    mesh = plsc.VectorSubcoreMesh(core_axis_name="c", subcore_axis_name="s")

    @functools.partial(
        pl.kernel,
        mesh=mesh,
        out_type=jax.ShapeDtypeStruct((n_rows * _C,), jnp.float32),
        scratch_types=(
            [pltpu.VMEM((_C,), jnp.int32)]
            + [pltpu.VMEM((_CHUNK,), jnp.float32) for _ in range(2 * _NB)]
            + [pltpu.SemaphoreType.DMA for _ in range(2 * _NB)]
        ),
        compiler_params=pltpu.CompilerParams(needs_layout_passes=False),
    )
    def k(x_hbm, idx_hbm, out_hbm, idx_v, *bufs_and_sems):
        ins = bufs_and_sems[:_NB]
        outs = bufs_and_sems[_NB:2 * _NB]
        lsems = bufs_and_sems[2 * _NB:3 * _NB]
        ssems = bufs_and_sems[3 * _NB:]
        wid = lax.axis_index("s") * 2 + lax.axis_index("c")
        base = wid * _CHUNK
        pltpu.sync_copy(idx_hbm, idx_v)
        idx_regs = [idx_v[pl.ds(g * _L, _L)] for g in range(_C // _L)]

        def start_load(ci, b):
            pltpu.async_copy(
                x_hbm.at[pl.ds(base + ci * (_NW * _CHUNK), _CHUNK)],
                ins[b], lsems[b])

        def start_store(ci, b):
            pltpu.async_copy(
                outs[b],
                out_hbm.at[pl.ds(base + ci * (_NW * _CHUNK), _CHUNK)],
                ssems[b])

        def wait_load(b):
            pltpu.make_async_copy(x_hbm.at[pl.ds(0, _CHUNK)], ins[b],
                                  lsems[b]).wait()

        def wait_store(b):
            pltpu.make_async_copy(outs[b], out_hbm.at[pl.ds(0, _CHUNK)],
                                  ssems[b]).wait()

        def permute_chunk(b):
            @plsc.parallel_loop(0, _R, unroll=8)
            def _(r):
                rb = r * _C
                src = ins[b].at[pl.ds(rb, _C)]
                for g in range(_C // _L):
                    outs[b][pl.ds(rb + g * _L, _L)] = plsc.load_gather(
                        src, [idx_regs[g]])

        for b in range(_NB):
            start_load(b, b)

        def group_body(i, _):
            c0 = _NB * i
            for b in range(_NB):
                wait_load(b)

                @pl.when(i > 0)
                def _():
                    wait_store(b)

                permute_chunk(b)
                start_store(c0 + b, b)

                @pl.when(i + 1 < n_groups)
                def _():
                    start_load(c0 + _NB + b, b)
            return 0

        lax.fori_loop(0, n_groups, group_body, 0)
        for b in range(_NB):
            wait_store(b)

    return k(xf, indexes)


def kernel(x, indexes):
    b, s, c = x.shape
    n_rows = b * s
    out = _sc_permute(x.reshape(n_rows * c), indexes, n_rows)
    return out.reshape(b, s, c)


# speedup vs baseline: 1.0054x; 1.0054x over previous
import functools

import jax
import jax.numpy as jnp
from jax import lax
from jax.experimental import pallas as pl
from jax.experimental.pallas import tpu as pltpu
from jax.experimental.pallas import tpu_sc as plsc

_L = 16
_C = 128
_NW = 32
_R = 64
_CHUNK = _R * _C
_NB = 4


def _sc_permute(xf, indexes, n_rows):
    rows_per_w = n_rows // _NW
    n_chunks = rows_per_w // _R
    n_groups = n_chunks // _NB
    mesh = plsc.VectorSubcoreMesh(core_axis_name="c", subcore_axis_name="s")

    @functools.partial(
        pl.kernel,
        mesh=mesh,
        out_type=jax.ShapeDtypeStruct((n_rows * _C,), jnp.float32),
        scratch_types=(
            [pltpu.VMEM((_C,), jnp.int32)]
            + [pltpu.VMEM((_CHUNK,), jnp.float32) for _ in range(2 * _NB)]
            + [pltpu.SemaphoreType.DMA for _ in range(2 * _NB)]
        ),
        compiler_params=pltpu.CompilerParams(needs_layout_passes=False),
    )
    def k(x_hbm, idx_hbm, out_hbm, idx_v, *bufs_and_sems):
        ins = bufs_and_sems[:_NB]
        outs = bufs_and_sems[_NB:2 * _NB]
        lsems = bufs_and_sems[2 * _NB:3 * _NB]
        ssems = bufs_and_sems[3 * _NB:]
        wid = lax.axis_index("s") * 2 + lax.axis_index("c")
        base = wid * _CHUNK
        pltpu.sync_copy(idx_hbm, idx_v)
        idx_regs = [idx_v[pl.ds(g * _L, _L)] for g in range(_C // _L)]

        def start_load(ci, b):
            pltpu.async_copy(
                x_hbm.at[pl.ds(base + ci * (_NW * _CHUNK), _CHUNK)],
                ins[b], lsems[b])

        def start_store(ci, b):
            pltpu.async_copy(
                outs[b],
                out_hbm.at[pl.ds(base + ci * (_NW * _CHUNK), _CHUNK)],
                ssems[b])

        def wait_load(b):
            pltpu.make_async_copy(x_hbm.at[pl.ds(0, _CHUNK)], ins[b],
                                  lsems[b]).wait()

        def wait_store(b):
            pltpu.make_async_copy(outs[b], out_hbm.at[pl.ds(0, _CHUNK)],
                                  ssems[b]).wait()

        def permute_chunk(b):
            @plsc.parallel_loop(0, _R, unroll=8)
            def _(r):
                rb = r * _C
                src = ins[b].at[pl.ds(rb, _C)]
                for g in range(_C // _L):
                    outs[b][pl.ds(rb + g * _L, _L)] = plsc.load_gather(
                        src, [idx_regs[g]])

        for b in range(_NB):
            start_load(b, b)

        def group_body(i, _):
            c0 = _NB * i
            for b in range(_NB):
                wait_load(b)

                @pl.when(i > 0)
                def _():
                    wait_store(b)

                permute_chunk(b)
                start_store(c0 + b, b)

                @pl.when(i + 1 < n_groups)
                def _():
                    start_load(c0 + _NB + b, b)
            return 0

        lax.fori_loop(0, n_groups, group_body, 0)
        for b in range(_NB):
            wait_store(b)

    return k(xf, indexes)


def kernel(x, indexes):
    b, s, c = x.shape
    n_rows = b * s
    out = _sc_permute(x.reshape(n_rows * c), indexes, n_rows)
    return out.reshape(b, s, c)
